# baseline (speedup 1.0000x reference)
import jax
import jax.numpy as jnp
from jax.experimental import pallas as pl
from jax.experimental.pallas import tpu as pltpu

N = 512
K = 32
G = 16
TG = 512


def _net_kernel(x_ref, a_ref, e_ref,
                Ws1_ref, bs1_ref, Wblk1_ref, bai1_ref, bao1_ref,
                Wn1_ref, bn1_ref, be1_ref,
                Ws2_ref, bs2_ref, Wblk2_ref, bai2_ref, bao2_ref,
                Wn2_ref, bn2_ref, Wd_ref, bd_ref,
                y_ref,
                u_s, m_in_s, e1_s, e1T_s, eT_s):
    x = x_ref[...]
    eT_s[...] = e_ref[...].T
    onesblk = (jax.lax.broadcasted_iota(jnp.int32, (K, G * K), 0)
               == jax.lax.broadcasted_iota(jnp.int32, (K, G * K), 1) % K
               ).astype(jnp.float32)
    mask2d = (jax.lax.broadcasted_iota(jnp.int32, (G * K, G), 0) // K
              == jax.lax.broadcasted_iota(jnp.int32, (G * K, G), 1)
              ).astype(jnp.float32)

    def xenet(x_arr, e_src, eT_src, Ws, bs, Wblk, bai, bao, be, want_e):
        f_in = x_arr.shape[1]
        nr = 3 if want_e else 2
        u_s[...] = jnp.dot(x_arr, Ws[:f_in, :],
                           preferred_element_type=jnp.float32) + bs
        vT = jax.lax.dot_general(Ws[f_in:2 * f_in, :], x_arr,
                                 (((0,), (1,)), ((), ())),
                                 preferred_element_type=jnp.float32)
        p = Ws[2 * f_in:2 * f_in + 1, :].reshape(1, K, 1)
        q = Ws[2 * f_in + 1:2 * f_in + 2, :].reshape(1, K, 1)

        def body(gi, m_out):
            i0 = gi * TG
            e_blk = e_src[pl.ds(i0, TG), :]
            et_blk = eT_src[pl.ds(i0, TG), :]
            a_blk = a_ref[pl.ds(i0, TG), :]
            u_blk = u_s[pl.ds(i0, TG), :]
            mk = (a_blk != 0.0).astype(jnp.float32)
            for s in range(TG // G):
                sl = slice(s * G, (s + 1) * G)
                z = (u_blk[sl][:, :, None] + vT[None, :, :]
                     + p * e_blk[sl][:, None, :] + q * et_blk[sl][:, None, :])
                t3 = jnp.maximum(z, 0.0)
                ts = t3.reshape(G * K, N)
                Rm = jnp.dot(Wblk, ts,
                             preferred_element_type=jnp.float32)
                ai = jax.nn.sigmoid(Rm[0:G, :] + bai)
                ao = jax.nn.sigmoid(Rm[G:2 * G, :] + bao)
                mks = mk[s * G:(s + 1) * G, :]
                wi = mks * ai
                wo = mks * ao
                Mi = jax.lax.dot_general(ts, wi, (((1,), (1,)), ((), ())),
                                         preferred_element_type=jnp.float32)
                m_in_blk = jnp.sum(Mi * mask2d, axis=1).reshape(G, K)
                m_in_s[pl.ds(i0 + s * G, G), :] = m_in_blk
                Pwo = (t3 * wo[:, None, :]).reshape(G * K, N)
                m_out = m_out + jnp.dot(onesblk, Pwo,
                                        preferred_element_type=jnp.float32)
                if want_e:
                    e1_s[pl.ds(i0 + s * G, G), :] = Rm[2 * G:3 * G, :] + be
            return m_out

        m_out = jax.lax.fori_loop(0, N // TG, body,
                                  jnp.zeros((K, N), jnp.float32))
        return m_out

    def node_update(x_arr, m_out, Wn, bn):
        f_in = x_arr.shape[1]
        out = jnp.dot(x_arr, Wn[:f_in, :],
                      preferred_element_type=jnp.float32)
        out = out + jnp.dot(m_in_s[...], Wn[f_in:f_in + K, :],
                            preferred_element_type=jnp.float32)
        out = out + jax.lax.dot_general(m_out, Wn[f_in + K:f_in + 2 * K, :],
                                        (((0,), (0,)), ((), ())),
                                        preferred_element_type=jnp.float32)
        return out + bn

    m_out1 = xenet(x, e_ref, eT_s,
                   Ws1_ref[...], bs1_ref[...], Wblk1_ref[...],
                   bai1_ref[...], bao1_ref[...], be1_ref[...], True)
    x1 = node_update(x, m_out1, Wn1_ref[...], bn1_ref[...])
    e1T_s[...] = e1_s[...].T

    m_out2 = xenet(x1, e1_s, e1T_s,
                   Ws2_ref[...], bs2_ref[...], Wblk2_ref[...],
                   bai2_ref[...], bao2_ref[...], None, False)
    x2 = node_update(x1, m_out2, Wn2_ref[...], bn2_ref[...])

    y_ref[...] = jnp.dot(x2, Wd_ref[...],
                         preferred_element_type=jnp.float32) + bd_ref[...]


def _blockdiag(W3):
    nr = W3.shape[0]
    A = jnp.einsum('rk,ij->rijk', W3, jnp.eye(G, dtype=jnp.float32))
    return A.reshape(nr * G, G * K)


def kernel(x, a, e, Ws1, bs1, Wai1, bai1, Wao1, bao1, Wn1, bn1, We1, be1,
           Ws2, bs2, Wai2, bai2, Wao2, bao2, Wn2, bn2, We2, be2, Wd, bd):
    del We2, be2
    x0 = x[0]
    a0 = a[0]
    e0 = e.reshape(N, N)
    Wblk1 = _blockdiag(jnp.concatenate([Wai1.T, Wao1.T, We1.T], axis=0))
    Wblk2 = _blockdiag(jnp.concatenate([Wai2.T, Wao2.T], axis=0))
    args = (x0, a0, e0,
            Ws1, bs1.reshape(1, -1), Wblk1, bai1.reshape(1, 1),
            bao1.reshape(1, 1), Wn1, bn1.reshape(1, -1), be1.reshape(1, 1),
            Ws2, bs2.reshape(1, -1), Wblk2, bai2.reshape(1, 1),
            bao2.reshape(1, 1), Wn2, bn2.reshape(1, -1),
            Wd, bd.reshape(1, -1))
    out = pl.pallas_call(
        _net_kernel,
        out_shape=jax.ShapeDtypeStruct((N, 240), jnp.float32),
        scratch_shapes=[
            pltpu.VMEM((N, K), jnp.float32),
            pltpu.VMEM((N, K), jnp.float32),
            pltpu.VMEM((N, N), jnp.float32),
            pltpu.VMEM((N, N), jnp.float32),
            pltpu.VMEM((N, N), jnp.float32),
        ],
    )(*args)
    return out[None]


# speedup vs baseline: 1.0187x; 1.0187x over previous
import jax
import jax.numpy as jnp
from jax.experimental import pallas as pl
from jax.experimental.pallas import tpu as pltpu

N = 512
K = 32
G = 32
TG = 512


def _net_kernel(x_ref, a_ref, e_ref,
                Ws1_ref, bs1_ref, Wblk1_ref, bai1_ref, bao1_ref,
                Wn1_ref, bn1_ref, be1_ref,
                Ws2_ref, bs2_ref, Wblk2_ref, bai2_ref, bao2_ref,
                Wn2_ref, bn2_ref, Wd_ref, bd_ref,
                y_ref,
                u_s, m_in_s, e1_s, e1T_s, eT_s):
    x = x_ref[...]
    eT_s[...] = e_ref[...].T
    onesblk = (jax.lax.broadcasted_iota(jnp.int32, (K, G * K), 0)
               == jax.lax.broadcasted_iota(jnp.int32, (K, G * K), 1) % K
               ).astype(jnp.float32)
    mask2d = (jax.lax.broadcasted_iota(jnp.int32, (G * K, G), 0) // K
              == jax.lax.broadcasted_iota(jnp.int32, (G * K, G), 1)
              ).astype(jnp.float32)

    def xenet(x_arr, e_src, eT_src, Ws, bs, Wblk, bai, bao, be, want_e):
        f_in = x_arr.shape[1]
        nr = 3 if want_e else 2
        u_s[...] = jnp.dot(x_arr, Ws[:f_in, :],
                           preferred_element_type=jnp.float32) + bs
        vT = jax.lax.dot_general(Ws[f_in:2 * f_in, :], x_arr,
                                 (((0,), (1,)), ((), ())),
                                 preferred_element_type=jnp.float32)
        p = Ws[2 * f_in:2 * f_in + 1, :].reshape(1, K, 1)
        q = Ws[2 * f_in + 1:2 * f_in + 2, :].reshape(1, K, 1)

        def body(gi, m_out):
            i0 = gi * TG
            e_blk = e_src[pl.ds(i0, TG), :]
            et_blk = eT_src[pl.ds(i0, TG), :]
            a_blk = a_ref[pl.ds(i0, TG), :]
            u_blk = u_s[pl.ds(i0, TG), :]
            mk = (a_blk != 0.0).astype(jnp.float32)
            for s in range(TG // G):
                sl = slice(s * G, (s + 1) * G)
                z = (u_blk[sl][:, :, None] + vT[None, :, :]
                     + p * e_blk[sl][:, None, :] + q * et_blk[sl][:, None, :])
                t3 = jnp.maximum(z, 0.0)
                ts = t3.reshape(G * K, N)
                Rm = jnp.dot(Wblk, ts,
                             preferred_element_type=jnp.float32)
                ai = jax.nn.sigmoid(Rm[0:G, :] + bai)
                ao = jax.nn.sigmoid(Rm[G:2 * G, :] + bao)
                mks = mk[s * G:(s + 1) * G, :]
                wi = mks * ai
                wo = mks * ao
                Mi = jax.lax.dot_general(ts, wi, (((1,), (1,)), ((), ())),
                                         preferred_element_type=jnp.float32)
                m_in_blk = jnp.sum(Mi * mask2d, axis=1).reshape(G, K)
                m_in_s[pl.ds(i0 + s * G, G), :] = m_in_blk
                Pwo = (t3 * wo[:, None, :]).reshape(G * K, N)
                m_out = m_out + jnp.dot(onesblk, Pwo,
                                        preferred_element_type=jnp.float32)
                if want_e:
                    e1_s[pl.ds(i0 + s * G, G), :] = Rm[2 * G:3 * G, :] + be
            return m_out

        m_out = jax.lax.fori_loop(0, N // TG, body,
                                  jnp.zeros((K, N), jnp.float32))
        return m_out

    def node_update(x_arr, m_out, Wn, bn):
        f_in = x_arr.shape[1]
        out = jnp.dot(x_arr, Wn[:f_in, :],
                      preferred_element_type=jnp.float32)
        out = out + jnp.dot(m_in_s[...], Wn[f_in:f_in + K, :],
                            preferred_element_type=jnp.float32)
        out = out + jax.lax.dot_general(m_out, Wn[f_in + K:f_in + 2 * K, :],
                                        (((0,), (0,)), ((), ())),
                                        preferred_element_type=jnp.float32)
        return out + bn

    m_out1 = xenet(x, e_ref, eT_s,
                   Ws1_ref[...], bs1_ref[...], Wblk1_ref[...],
                   bai1_ref[...], bao1_ref[...], be1_ref[...], True)
    x1 = node_update(x, m_out1, Wn1_ref[...], bn1_ref[...])
    e1T_s[...] = e1_s[...].T

    m_out2 = xenet(x1, e1_s, e1T_s,
                   Ws2_ref[...], bs2_ref[...], Wblk2_ref[...],
                   bai2_ref[...], bao2_ref[...], None, False)
    x2 = node_update(x1, m_out2, Wn2_ref[...], bn2_ref[...])

    y_ref[...] = jnp.dot(x2, Wd_ref[...],
                         preferred_element_type=jnp.float32) + bd_ref[...]


def _blockdiag(W3):
    nr = W3.shape[0]
    A = jnp.einsum('rk,ij->rijk', W3, jnp.eye(G, dtype=jnp.float32))
    return A.reshape(nr * G, G * K)


def kernel(x, a, e, Ws1, bs1, Wai1, bai1, Wao1, bao1, Wn1, bn1, We1, be1,
           Ws2, bs2, Wai2, bai2, Wao2, bao2, Wn2, bn2, We2, be2, Wd, bd):
    del We2, be2
    x0 = x[0]
    a0 = a[0]
    e0 = e.reshape(N, N)
    Wblk1 = _blockdiag(jnp.concatenate([Wai1.T, Wao1.T, We1.T], axis=0))
    Wblk2 = _blockdiag(jnp.concatenate([Wai2.T, Wao2.T], axis=0))
    args = (x0, a0, e0,
            Ws1, bs1.reshape(1, -1), Wblk1, bai1.reshape(1, 1),
            bao1.reshape(1, 1), Wn1, bn1.reshape(1, -1), be1.reshape(1, 1),
            Ws2, bs2.reshape(1, -1), Wblk2, bai2.reshape(1, 1),
            bao2.reshape(1, 1), Wn2, bn2.reshape(1, -1),
            Wd, bd.reshape(1, -1))
    out = pl.pallas_call(
        _net_kernel,
        out_shape=jax.ShapeDtypeStruct((N, 240), jnp.float32),
        scratch_shapes=[
            pltpu.VMEM((N, K), jnp.float32),
            pltpu.VMEM((N, K), jnp.float32),
            pltpu.VMEM((N, N), jnp.float32),
            pltpu.VMEM((N, N), jnp.float32),
            pltpu.VMEM((N, N), jnp.float32),
        ],
    )(*args)
    return out[None]


# speedup vs baseline: 1.0241x; 1.0052x over previous
import jax
import jax.numpy as jnp
from jax.experimental import pallas as pl
from jax.experimental.pallas import tpu as pltpu

N = 512
K = 32
G = 32
TG = 512


def _net_kernel(x_ref, a_ref, e_ref,
                Ws1_ref, bs1_ref, Wblk1_ref, bai1_ref, bao1_ref,
                Wn1_ref, bn1_ref, be1_ref,
                Ws2_ref, bs2_ref, Wblk2_ref, bai2_ref, bao2_ref,
                Wn2_ref, bn2_ref, Wd_ref, bd_ref,
                y_ref,
                u_s, m_in_s, e1_s, e1T_s, eT_s):
    x = x_ref[...]
    eT_s[...] = e_ref[...].T
    onesblk = (jax.lax.broadcasted_iota(jnp.int32, (K, G * K), 0)
               == jax.lax.broadcasted_iota(jnp.int32, (K, G * K), 1) % K
               ).astype(jnp.float32)
    mask2d = (jax.lax.broadcasted_iota(jnp.int32, (G * K, G), 0) // K
              == jax.lax.broadcasted_iota(jnp.int32, (G * K, G), 1)
              ).astype(jnp.float32)

    def xenet(x_arr, e_src, eT_src, Ws, bs, Wblk, bai, bao, be, want_e):
        f_in = x_arr.shape[1]
        u_s[...] = jnp.dot(x_arr, Ws[:f_in, :],
                           preferred_element_type=jnp.float32) + bs
        vT = jax.lax.dot_general(Ws[f_in:2 * f_in, :], x_arr,
                                 (((0,), (1,)), ((), ())),
                                 preferred_element_type=jnp.float32)
        p = Ws[2 * f_in:2 * f_in + 1, :].reshape(1, K, 1)
        q = Ws[2 * f_in + 1:2 * f_in + 2, :].reshape(1, K, 1)

        def body(gi, m_out):
            i0 = gi * TG
            e_blk = e_src[pl.ds(i0, TG), :]
            et_blk = eT_src[pl.ds(i0, TG), :]
            a_blk = a_ref[pl.ds(i0, TG), :]
            u_blk = u_s[pl.ds(i0, TG), :]
            mk = (a_blk != 0.0).astype(jnp.float32)
            for s in range(TG // G):
                sl = slice(s * G, (s + 1) * G)
                z = (u_blk[sl][:, :, None] + vT[None, :, :]
                     + p * e_blk[sl][:, None, :] + q * et_blk[sl][:, None, :])
                t3 = jnp.maximum(z, 0.0)
                ts = t3.reshape(G * K, N)
                Rm = jnp.dot(Wblk, ts,
                             preferred_element_type=jnp.float32)
                ai = jax.nn.sigmoid(Rm[0:G, :] + bai)
                ao = jax.nn.sigmoid(Rm[G:2 * G, :] + bao)
                mks = mk[s * G:(s + 1) * G, :]
                wi = mks * ai
                wo = mks * ao
                Mi = jax.lax.dot_general(ts, wi, (((1,), (1,)), ((), ())),
                                         preferred_element_type=jnp.float32)
                m_in_blk = jnp.sum(Mi * mask2d, axis=1).reshape(G, K)
                m_in_s[pl.ds(i0 + s * G, G), :] = m_in_blk
                Pwo = (t3 * wo[:, None, :]).reshape(G * K, N)
                m_out = m_out + jnp.dot(onesblk, Pwo,
                                        preferred_element_type=jnp.float32)
                if want_e:
                    e1_s[pl.ds(i0 + s * G, G), :] = Rm[2 * G:3 * G, :] + be
            return m_out

        return body(0, jnp.zeros((K, N), jnp.float32))

    def node_update(x_arr, m_out, Wn, bn):
        f_in = x_arr.shape[1]
        out = jnp.dot(x_arr, Wn[:f_in, :],
                      preferred_element_type=jnp.float32)
        out = out + jnp.dot(m_in_s[...], Wn[f_in:f_in + K, :],
                            preferred_element_type=jnp.float32)
        out = out + jax.lax.dot_general(m_out, Wn[f_in + K:f_in + 2 * K, :],
                                        (((0,), (0,)), ((), ())),
                                        preferred_element_type=jnp.float32)
        return out + bn

    m_out1 = xenet(x, e_ref, eT_s,
                   Ws1_ref[...], bs1_ref[...], Wblk1_ref[...],
                   bai1_ref[...], bao1_ref[...], be1_ref[...], True)
    x1 = node_update(x, m_out1, Wn1_ref[...], bn1_ref[...])
    e1T_s[...] = e1_s[...].T

    m_out2 = xenet(x1, e1_s, e1T_s,
                   Ws2_ref[...], bs2_ref[...], Wblk2_ref[...],
                   bai2_ref[...], bao2_ref[...], None, False)
    x2 = node_update(x1, m_out2, Wn2_ref[...], bn2_ref[...])

    y_ref[...] = jnp.dot(x2, Wd_ref[...],
                         preferred_element_type=jnp.float32) + bd_ref[...]


def _blockdiag(W3):
    nr = W3.shape[0]
    A = jnp.einsum('rk,ij->rijk', W3, jnp.eye(G, dtype=jnp.float32))
    return A.reshape(nr * G, G * K)


def kernel(x, a, e, Ws1, bs1, Wai1, bai1, Wao1, bao1, Wn1, bn1, We1, be1,
           Ws2, bs2, Wai2, bai2, Wao2, bao2, Wn2, bn2, We2, be2, Wd, bd):
    del We2, be2
    x0 = x[0]
    a0 = a[0]
    e0 = e.reshape(N, N)
    Wblk1 = _blockdiag(jnp.concatenate([Wai1.T, Wao1.T, We1.T], axis=0))
    Wblk2 = _blockdiag(jnp.concatenate([Wai2.T, Wao2.T], axis=0))
    args = (x0, a0, e0,
            Ws1, bs1.reshape(1, -1), Wblk1, bai1.reshape(1, 1),
            bao1.reshape(1, 1), Wn1, bn1.reshape(1, -1), be1.reshape(1, 1),
            Ws2, bs2.reshape(1, -1), Wblk2, bai2.reshape(1, 1),
            bao2.reshape(1, 1), Wn2, bn2.reshape(1, -1),
            Wd, bd.reshape(1, -1))
    out = pl.pallas_call(
        _net_kernel,
        out_shape=jax.ShapeDtypeStruct((N, 240), jnp.float32),
        scratch_shapes=[
            pltpu.VMEM((N, K), jnp.float32),
            pltpu.VMEM((N, K), jnp.float32),
            pltpu.VMEM((N, N), jnp.float32),
            pltpu.VMEM((N, N), jnp.float32),
            pltpu.VMEM((N, N), jnp.float32),
        ],
    )(*args)
    return out[None]
